# Initial kernel scaffold; baseline (speedup 1.0000x reference)
import jax
import jax.numpy as jnp
from jax.experimental import pallas as pl


def kernel(x, edge_index, W0, b0, W1, b1, W2, b2):
    raise NotImplementedError("write your pallas kernel here")


# speedup vs baseline: 12.7537x; 12.7537x over previous
import functools

import jax
import jax.numpy as jnp
from jax import lax
from jax.experimental import pallas as pl
from jax.experimental.pallas import tpu as pltpu
from jax.experimental.pallas import tpu_sc as plsc

N = 10000
E = 320000
D = 128
NC = 2
NS = 16
NW = NC * NS
C = 80
EPW = E // NW
NCH = EPW // C
GC = 5
NG = NCH // GC
RPT = 624
REM = N - NS * RPT

_mesh = plsc.VectorSubcoreMesh(core_axis_name="c", subcore_axis_name="s")


def _row_base(idx):
    return pl.multiple_of(idx, 8)


def _init_acc(src_hbm, acc_sh, sid):
    b = _row_base(sid * RPT)
    pltpu.sync_copy(src_hbm.at[pl.ds(b, RPT)], acc_sh.at[pl.ds(b, RPT)])

    @pl.when(sid == 0)
    def _():
        pltpu.sync_copy(src_hbm.at[pl.ds(NS * RPT, REM)],
                        acc_sh.at[pl.ds(NS * RPT, REM)])


def _init_zero(zeros_hbm, acc_sh, sid):
    b = _row_base(sid * RPT)
    pltpu.sync_copy(zeros_hbm, acc_sh.at[pl.ds(b, RPT)])

    @pl.when(sid == 0)
    def _():
        pltpu.sync_copy(zeros_hbm.at[pl.ds(0, REM)],
                        acc_sh.at[pl.ds(NS * RPT, REM)])


def _read_acc(acc_sh, out_hbm, cid, sid):
    b_sh = _row_base(sid * RPT)
    b_out = _row_base(cid * N + sid * RPT)
    pltpu.sync_copy(acc_sh.at[pl.ds(b_sh, RPT)], out_hbm.at[pl.ds(b_out, RPT)])

    @pl.when(sid == 0)
    def _():
        b2 = _row_base(cid * N + NS * RPT)
        pltpu.sync_copy(acc_sh.at[pl.ds(NS * RPT, REM)],
                        out_hbm.at[pl.ds(b2, REM)])


@functools.partial(
    pl.kernel,
    out_type=jax.ShapeDtypeStruct((2 * N, D), jnp.float32),
    mesh=_mesh,
    scratch_types=[
        pltpu.VMEM((GC, C), jnp.int32),
        pltpu.VMEM((GC, C), jnp.int32),
        pltpu.VMEM((C, D), jnp.float32),
        pltpu.VMEM((C, D), jnp.float32),
        pltpu.VMEM_SHARED((N, D), jnp.float32),
        pltpu.SemaphoreType.DMA,
        pltpu.SemaphoreType.DMA,
    ],
)
def _agg_kernel(src_hbm, dst_hbm, hs_hbm, zeros_hbm, out_hbm,
                src_v, dst_v, rows_a, rows_b, acc_sh, sem_a, sem_b):
    cid = lax.axis_index("c")
    sid = lax.axis_index("s")
    wid = sid * NC + cid

    @pl.when(cid == 0)
    def _():
        _init_acc(hs_hbm, acc_sh, sid)

    @pl.when(cid != 0)
    def _():
        _init_zero(zeros_hbm, acc_sh, sid)

    plsc.subcore_barrier()

    def group(g, carry):
        pltpu.sync_copy(src_hbm.at[wid, g], src_v)
        pltpu.sync_copy(dst_hbm.at[wid, g], dst_v)
        pltpu.async_copy(hs_hbm.at[src_v.at[0]], rows_a, sem_a)
        for k in range(GC):
            rows, sem = (rows_a, sem_a) if k % 2 == 0 else (rows_b, sem_b)
            pltpu.make_async_copy(hs_hbm.at[src_v.at[k]], rows, sem).wait()
            if k + 1 < GC:
                nrows, nsem = (rows_b, sem_b) if k % 2 == 0 else (rows_a, sem_a)
                pltpu.async_copy(hs_hbm.at[src_v.at[k + 1]], nrows, nsem)
            pltpu.sync_copy(rows, acc_sh.at[dst_v.at[k]], add=True)
        return carry

    lax.fori_loop(0, NG, group, 0)
    plsc.subcore_barrier()
    _read_acc(acc_sh, out_hbm, cid, sid)


BM = 1000
G = N // BM


def _dinv(dega, degb):
    return lax.rsqrt(dega[:, 0:1] + degb[:, 0:1])


def _pre_body(h_ref, w_ref, dega_ref, degb_ref, o_ref):
    o_ref[...] = _dinv(dega_ref[...], degb_ref[...]) * jnp.dot(
        h_ref[...], w_ref[...], preferred_element_type=jnp.float32)


def _post_body(a0_ref, a1_ref, dega_ref, degb_ref, b_ref, flag_ref, id_ref,
               o_ref):
    dinv = _dinv(dega_ref[...], degb_ref[...])
    raw = dinv * (a0_ref[...] + a1_ref[...]) + b_ref[...]
    o_ref[...] = jnp.where(flag_ref[...] > 0.0,
                           jnp.maximum(raw, 0.0) + id_ref[...], raw)


def _deg_specs():
    return [
        pl.BlockSpec((BM, D), lambda i: (i, 0)),
        pl.BlockSpec((BM, D), lambda i: (i + G, 0)),
    ]


def _pre_call(h, W, degp):
    return pl.pallas_call(
        _pre_body,
        grid=(G,),
        in_specs=[
            pl.BlockSpec((BM, D), lambda i: (i, 0)),
            pl.BlockSpec((D, D), lambda i: (0, 0)),
            *_deg_specs(),
        ],
        out_specs=pl.BlockSpec((BM, D), lambda i: (i, 0)),
        out_shape=jax.ShapeDtypeStruct((N, D), jnp.float32),
    )(h, W, degp, degp)


def _post_call(agg, degp, b, flag, ident):
    return pl.pallas_call(
        _post_body,
        grid=(G,),
        in_specs=[
            pl.BlockSpec((BM, D), lambda i: (i, 0)),
            pl.BlockSpec((BM, D), lambda i: (i + G, 0)),
            *_deg_specs(),
            pl.BlockSpec((1, D), lambda i: (0, 0)),
            pl.BlockSpec((1, D), lambda i: (0, 0)),
            pl.BlockSpec((BM, D), lambda i: (i, 0)),
        ],
        out_specs=pl.BlockSpec((BM, D), lambda i: (i, 0)),
        out_shape=jax.ShapeDtypeStruct((N, D), jnp.float32),
    )(agg, agg, degp, degp, b, flag, ident)


def kernel(x, edge_index, W0, b0, W1, b1, W2, b2):
    ei = edge_index.astype(jnp.int32)
    src2 = ei[0].reshape(NW, NG, GC, C)
    dst2 = ei[1].reshape(NW, NG, GC, C)
    zeros_h = jnp.zeros((RPT, D), jnp.float32)
    ones_n = jnp.ones((N, D), jnp.float32)

    degp = _agg_kernel(src2, dst2, ones_n, zeros_h)

    W2p = jnp.pad(W2, ((0, 0), (0, D - W2.shape[1])))
    b2p = jnp.pad(b2, (0, D - b2.shape[0]))
    Wstk = jnp.stack([W0, W1, W2p])
    bstk = jnp.stack([b0, b1, b2p]).reshape(3, 1, D)
    flagstk = jnp.broadcast_to(
        jnp.array([1.0, 1.0, 0.0], jnp.float32).reshape(3, 1, 1), (3, 1, D))

    def layer(h, xs):
        W, b, flag = xs
        hs = _pre_call(h, W, degp)
        agg = _agg_kernel(src2, dst2, hs, zeros_h)
        h2 = _post_call(agg, degp, b, flag, h)
        return h2, None

    h, _ = lax.scan(layer, x, (Wstk, bstk, flagstk))
    return h[:, :64]


# speedup vs baseline: 12.9981x; 1.0192x over previous
import functools

import jax
import jax.numpy as jnp
from jax import lax
from jax.experimental import pallas as pl
from jax.experimental.pallas import tpu as pltpu
from jax.experimental.pallas import tpu_sc as plsc

N = 10000
E = 320000
D = 128
NC = 2
NS = 16
NW = NC * NS
C = 80
EPW = E // NW
NCH = EPW // C
GC = 5
NG = NCH // GC
RPT = 624
REM = N - NS * RPT

_mesh = plsc.VectorSubcoreMesh(core_axis_name="c", subcore_axis_name="s")


def _row_base(idx):
    return pl.multiple_of(idx, 8)


def _init_acc(src_hbm, acc_sh, sid):
    b = _row_base(sid * RPT)
    pltpu.sync_copy(src_hbm.at[pl.ds(b, RPT)], acc_sh.at[pl.ds(b, RPT)])

    @pl.when(sid == 0)
    def _():
        pltpu.sync_copy(src_hbm.at[pl.ds(NS * RPT, REM)],
                        acc_sh.at[pl.ds(NS * RPT, REM)])


def _init_zero(zeros_hbm, acc_sh, sid):
    b = _row_base(sid * RPT)
    pltpu.sync_copy(zeros_hbm, acc_sh.at[pl.ds(b, RPT)])

    @pl.when(sid == 0)
    def _():
        pltpu.sync_copy(zeros_hbm.at[pl.ds(0, REM)],
                        acc_sh.at[pl.ds(NS * RPT, REM)])


def _read_acc(acc_sh, out_hbm, cid, sid):
    b_sh = _row_base(sid * RPT)
    b_out = _row_base(cid * N + sid * RPT)
    pltpu.sync_copy(acc_sh.at[pl.ds(b_sh, RPT)], out_hbm.at[pl.ds(b_out, RPT)])

    @pl.when(sid == 0)
    def _():
        b2 = _row_base(cid * N + NS * RPT)
        pltpu.sync_copy(acc_sh.at[pl.ds(NS * RPT, REM)],
                        out_hbm.at[pl.ds(b2, REM)])


@functools.partial(
    pl.kernel,
    out_type=jax.ShapeDtypeStruct((2 * N, D), jnp.float32),
    mesh=_mesh,
    scratch_types=[
        pltpu.VMEM((GC, C), jnp.int32),
        pltpu.VMEM((GC, C), jnp.int32),
        pltpu.VMEM((C, D), jnp.float32),
        pltpu.VMEM((C, D), jnp.float32),
        pltpu.VMEM_SHARED((N, D), jnp.float32),
        pltpu.SemaphoreType.DMA,
        pltpu.SemaphoreType.DMA,
    ],
)
def _agg_kernel(src_hbm, dst_hbm, hs_hbm, zeros_hbm, out_hbm,
                src_v, dst_v, rows_a, rows_b, acc_sh, sem_a, sem_b):
    cid = lax.axis_index("c")
    sid = lax.axis_index("s")
    wid = sid * NC + cid

    @pl.when(cid == 0)
    def _():
        _init_acc(hs_hbm, acc_sh, sid)

    @pl.when(cid != 0)
    def _():
        _init_zero(zeros_hbm, acc_sh, sid)

    plsc.subcore_barrier()

    def group(g, carry):
        pltpu.sync_copy(src_hbm.at[wid, g], src_v)
        pltpu.sync_copy(dst_hbm.at[wid, g], dst_v)
        pltpu.async_copy(hs_hbm.at[src_v.at[0]], rows_a, sem_a)
        for k in range(GC):
            rows, sem = (rows_a, sem_a) if k % 2 == 0 else (rows_b, sem_b)
            pltpu.make_async_copy(hs_hbm.at[src_v.at[k]], rows, sem).wait()
            if k + 1 < GC:
                nrows, nsem = (rows_b, sem_b) if k % 2 == 0 else (rows_a, sem_a)
                pltpu.async_copy(hs_hbm.at[src_v.at[k + 1]], nrows, nsem)
            pltpu.sync_copy(rows, acc_sh.at[dst_v.at[k]], add=True)
        return carry

    lax.fori_loop(0, NG, group, 0)
    plsc.subcore_barrier()
    _read_acc(acc_sh, out_hbm, cid, sid)


BM = 1000
G = N // BM


def _dinv(dega, degb):
    return lax.rsqrt(dega[:, 0:1] + degb[:, 0:1])


def _pre_body(h_ref, w_ref, dega_ref, degb_ref, o_ref):
    o_ref[...] = _dinv(dega_ref[...], degb_ref[...]) * jnp.dot(
        h_ref[...], w_ref[...], preferred_element_type=jnp.float32)


def _post_body(a0_ref, a1_ref, dega_ref, degb_ref, b_ref, flag_ref, id_ref,
               wn_ref, h_ref, hsn_ref):
    dinv = _dinv(dega_ref[...], degb_ref[...])
    raw = dinv * (a0_ref[...] + a1_ref[...]) + b_ref[...]
    h = jnp.where(flag_ref[...] > 0.0,
                  jnp.maximum(raw, 0.0) + id_ref[...], raw)
    h_ref[...] = h
    hsn_ref[...] = dinv * jnp.dot(h, wn_ref[...],
                                  preferred_element_type=jnp.float32)


def _deg_specs():
    return [
        pl.BlockSpec((BM, D), lambda i: (i, 0)),
        pl.BlockSpec((BM, D), lambda i: (i + G, 0)),
    ]


def _pre_call(h, W, degp):
    return pl.pallas_call(
        _pre_body,
        grid=(G,),
        in_specs=[
            pl.BlockSpec((BM, D), lambda i: (i, 0)),
            pl.BlockSpec((D, D), lambda i: (0, 0)),
            *_deg_specs(),
        ],
        out_specs=pl.BlockSpec((BM, D), lambda i: (i, 0)),
        out_shape=jax.ShapeDtypeStruct((N, D), jnp.float32),
    )(h, W, degp, degp)


def _post_call(agg, degp, b, flag, ident, Wn):
    return pl.pallas_call(
        _post_body,
        grid=(G,),
        in_specs=[
            pl.BlockSpec((BM, D), lambda i: (i, 0)),
            pl.BlockSpec((BM, D), lambda i: (i + G, 0)),
            *_deg_specs(),
            pl.BlockSpec((1, D), lambda i: (0, 0)),
            pl.BlockSpec((1, D), lambda i: (0, 0)),
            pl.BlockSpec((BM, D), lambda i: (i, 0)),
            pl.BlockSpec((D, D), lambda i: (0, 0)),
        ],
        out_specs=[
            pl.BlockSpec((BM, D), lambda i: (i, 0)),
            pl.BlockSpec((BM, D), lambda i: (i, 0)),
        ],
        out_shape=[
            jax.ShapeDtypeStruct((N, D), jnp.float32),
            jax.ShapeDtypeStruct((N, D), jnp.float32),
        ],
    )(agg, agg, degp, degp, b, flag, ident, Wn)


def kernel(x, edge_index, W0, b0, W1, b1, W2, b2):
    ei = edge_index.astype(jnp.int32)
    src2 = ei[0].reshape(NW, NG, GC, C)
    dst2 = ei[1].reshape(NW, NG, GC, C)
    zeros_h = jnp.zeros((RPT, D), jnp.float32)
    ones_n = jnp.ones((N, D), jnp.float32)

    degp = _agg_kernel(src2, dst2, ones_n, zeros_h)

    W2p = jnp.pad(W2, ((0, 0), (0, D - W2.shape[1])))
    b2p = jnp.pad(b2, (0, D - b2.shape[0]))
    Wstk = jnp.stack([W1, W2p, W2p])
    bstk = jnp.stack([b0, b1, b2p]).reshape(3, 1, D)
    flagstk = jnp.broadcast_to(
        jnp.array([1.0, 1.0, 0.0], jnp.float32).reshape(3, 1, 1), (3, 1, D))

    hs0 = _pre_call(x, W0, degp)

    def layer(carry, xs):
        h, hs = carry
        Wn, b, flag = xs
        agg = _agg_kernel(src2, dst2, hs, zeros_h)
        h2, hsn = _post_call(agg, degp, b, flag, h, Wn)
        return (h2, hsn), None

    (h, _), _ = lax.scan(layer, (x, hs0), (Wstk, bstk, flagstk))
    return h[:, :64]
